# Initial kernel scaffold; baseline (speedup 1.0000x reference)
import jax
import jax.numpy as jnp
from jax.experimental import pallas as pl


def kernel(x, proj1, proj1_bias, proj2, proj2_bias, gate_w):
    raise NotImplementedError("write your pallas kernel here")


# speedup vs baseline: 1.8105x; 1.8105x over previous
import functools

import jax
import jax.numpy as jnp
from jax.experimental import pallas as pl
from jax.experimental.pallas import tpu as pltpu

_M_TILE = 1024
_H_TILE = 512


def _mlp_body(x_ref, w1_ref, b1_ref, w2_ref, b2_ref, o_ref):
    h_idx = pl.program_id(1)
    t = jax.lax.dot_general(
        x_ref[...], w1_ref[...],
        dimension_numbers=(((1,), (1,)), ((), ())),
        preferred_element_type=jnp.float32,
    )
    t = t + b1_ref[...]
    t = 0.5 * t * (1.0 + jax.lax.erf(t * 0.7071067811865476))
    contrib = jax.lax.dot_general(
        t.astype(jnp.bfloat16), w2_ref[...],
        dimension_numbers=(((1,), (1,)), ((), ())),
        preferred_element_type=jnp.float32,
    )

    @pl.when(h_idx == 0)
    def _init():
        o_ref[...] = contrib + b2_ref[...]

    @pl.when(h_idx != 0)
    def _accum():
        o_ref[...] += contrib


@functools.partial(jax.jit, static_argnames=("m_tile", "h_tile"))
def _fused_mlp(xm, w1, b1, w2, b2, m_tile=_M_TILE, h_tile=_H_TILE):
    m, embed = xm.shape
    hidden = w1.shape[0]
    grid = (m // m_tile, hidden // h_tile)
    return pl.pallas_call(
        _mlp_body,
        grid=grid,
        in_specs=[
            pl.BlockSpec((m_tile, embed), lambda i, j: (i, 0)),
            pl.BlockSpec((h_tile, embed), lambda i, j: (j, 0)),
            pl.BlockSpec((1, h_tile), lambda i, j: (0, j)),
            pl.BlockSpec((embed, h_tile), lambda i, j: (0, j)),
            pl.BlockSpec((1, embed), lambda i, j: (0, 0)),
        ],
        out_specs=pl.BlockSpec((m_tile, embed), lambda i, j: (i, 0)),
        out_shape=jax.ShapeDtypeStruct((m, embed), jnp.float32),
        compiler_params=pltpu.CompilerParams(
            dimension_semantics=("parallel", "arbitrary"),
        ),
    )(xm, w1, b1, w2, b2)


def kernel(x, proj1, proj1_bias, proj2, proj2_bias, gate_w):
    del gate_w
    length, n, embed = x.shape
    xm = x.reshape(length * n, embed).astype(jnp.bfloat16)
    out = _fused_mlp(
        xm,
        proj1.astype(jnp.bfloat16),
        proj1_bias.reshape(1, -1),
        proj2.astype(jnp.bfloat16),
        proj2_bias.reshape(1, -1),
    )
    return out.reshape(length, n, embed)


# speedup vs baseline: 2.0453x; 1.1297x over previous
import functools

import jax
import jax.numpy as jnp
from jax.experimental import pallas as pl
from jax.experimental.pallas import tpu as pltpu

_M_TILE = 1024
_H_TILE = 512


def _mlp_body(x_ref, w1_ref, b1_ref, w2_ref, b2_ref, o_ref):
    h_idx = pl.program_id(1)
    t = jax.lax.dot_general(
        x_ref[...], w1_ref[...].astype(jnp.bfloat16),
        dimension_numbers=(((1,), (1,)), ((), ())),
        preferred_element_type=jnp.float32,
    )
    t = t + b1_ref[...]
    t = 0.5 * t * (1.0 + jax.lax.erf(t * 0.7071067811865476))
    contrib = jax.lax.dot_general(
        t.astype(jnp.bfloat16), w2_ref[...].astype(jnp.bfloat16),
        dimension_numbers=(((1,), (1,)), ((), ())),
        preferred_element_type=jnp.float32,
    )

    @pl.when(h_idx == 0)
    def _init():
        o_ref[...] = contrib + b2_ref[...]

    @pl.when(h_idx != 0)
    def _accum():
        o_ref[...] += contrib


@functools.partial(jax.jit, static_argnames=("m_tile", "h_tile"))
def _fused_mlp(xm, w1, b1, w2, b2, m_tile=_M_TILE, h_tile=_H_TILE):
    m, embed = xm.shape
    hidden = w1.shape[0]
    grid = (m // m_tile, hidden // h_tile)
    return pl.pallas_call(
        _mlp_body,
        grid=grid,
        in_specs=[
            pl.BlockSpec((m_tile, embed), lambda i, j: (i, 0)),
            pl.BlockSpec((h_tile, embed), lambda i, j: (j, 0)),
            pl.BlockSpec((1, h_tile), lambda i, j: (0, j)),
            pl.BlockSpec((embed, h_tile), lambda i, j: (0, j)),
            pl.BlockSpec((1, embed), lambda i, j: (0, 0)),
        ],
        out_specs=pl.BlockSpec((m_tile, embed), lambda i, j: (i, 0)),
        out_shape=jax.ShapeDtypeStruct((m, embed), jnp.float32),
        compiler_params=pltpu.CompilerParams(
            dimension_semantics=("parallel", "arbitrary"),
        ),
    )(xm, w1, b1, w2, b2)


def kernel(x, proj1, proj1_bias, proj2, proj2_bias, gate_w):
    del gate_w
    length, n, embed = x.shape
    xm = x.reshape(length * n, embed).astype(jnp.bfloat16)
    out = _fused_mlp(
        xm,
        proj1,
        proj1_bias.reshape(1, -1),
        proj2,
        proj2_bias.reshape(1, -1),
    )
    return out.reshape(length, n, embed)
